# baseline (speedup 1.0000x reference)
import functools

import jax
import jax.numpy as jnp
from jax import lax
from jax.experimental import pallas as pl
from jax.experimental.pallas import tpu as pltpu
from jax.experimental.pallas import tpu_sc as plsc

C = 128
N = 10000
K = 32
OUT = 128
GROUPS = 4

NC = 2
NS = 16
NW = NC * NS
CPT = C // NW
WPT = CPT // 2
NB = 400
NBLK = N // NB
GPB = NB // 16


@functools.cache
def _build_sc_aggregate():
    mesh = plsc.VectorSubcoreMesh(
        core_axis_name="c", subcore_axis_name="s",
        num_cores=NC, num_subcores=NS)
    return pl.kernel(
        _sc_aggregate_body,
        out_type=jax.ShapeDtypeStruct((C, N), jnp.float32),
        mesh=mesh,
        compiler_params=pltpu.CompilerParams(
            use_tc_tiling_on_sc=False, needs_layout_passes=False),
        scratch_types=[
            pltpu.VMEM((WPT, N), jnp.int32),
            pltpu.VMEM((CPT, N), jnp.float32),
            pltpu.VMEM((2, K, NB), jnp.int32),
            pltpu.VMEM((2, K, NB), jnp.int32),
            pltpu.SemaphoreType.DMA,
            pltpu.SemaphoreType.DMA,
        ],
    )


def _sc_aggregate_body(x_hbm, ep_hbm, m_hbm, x_v, m_v, e_v0, e_v1, sem0, sem1):
    wid = lax.axis_index("s") * NC + lax.axis_index("c")
    c0 = wid * CPT
    pltpu.async_copy(ep_hbm.at[0], e_v0, sem0)
    pltpu.sync_copy(x_hbm.at[pl.ds(wid * WPT, WPT), :], x_v)

    wsplat = [jnp.full((16,), w, jnp.int32) for w in range(WPT)]

    def gather_bf16(widx, nidx):
        word = plsc.load_gather(x_v, [wsplat[widx], nidx])
        return plsc.bitcast(word, jnp.bfloat16)

    def compute_block(blk, e_v):
        @plsc.parallel_loop(0, GPB, unroll=5)
        def g_body(g):
            base = blk * NB + g * 16
            lbase = g * 16

            def edge_vectors(k):
                return (e_v[0, k, pl.ds(lbase, 16)],
                        e_v[1, k, pl.ds(lbase, 16)])

            vj0, vi0 = edge_vectors(0)
            accs = []
            for w in range(WPT):
                accs.append(gather_bf16(w, vj0) - gather_bf16(w, vi0))
            for k in range(1, K):
                vj, vi = edge_vectors(k)
                for w in range(WPT):
                    accs[w] = jnp.maximum(
                        accs[w], gather_bf16(w, vj) - gather_bf16(w, vi))
            for w in range(WPT):
                a_even, a_odd = plsc.unpack(
                    accs[w], format=plsc.PackFormat.INTERLEAVED)
                m_v[2 * w, pl.ds(base, 16)] = a_even
                m_v[2 * w + 1, pl.ds(base, 16)] = a_odd

    def blk_body(blk, _):
        nxt = blk + 1

        @pl.when(blk % 2 == 0)
        def _even():
            @pl.when(nxt < NBLK)
            def _():
                pltpu.async_copy(ep_hbm.at[nxt], e_v1, sem1)
            pltpu.make_async_copy(ep_hbm.at[blk], e_v0, sem0).wait()
            compute_block(blk, e_v0)

        @pl.when(blk % 2 == 1)
        def _odd():
            @pl.when(nxt < NBLK)
            def _():
                pltpu.async_copy(ep_hbm.at[nxt], e_v0, sem0)
            pltpu.make_async_copy(ep_hbm.at[blk], e_v1, sem1).wait()
            compute_block(blk, e_v1)

        return 0

    lax.fori_loop(0, NBLK, blk_body, 0)
    pltpu.sync_copy(m_v, m_hbm.at[pl.ds(c0, CPT), :])


def _conv_body(x_ref, m_ref, wx_ref, wm_ref, b_ref, o_ref):
    acc = jnp.dot(wx_ref[...], x_ref[...], preferred_element_type=jnp.float32)
    acc = acc + jnp.dot(wm_ref[...], m_ref[...], preferred_element_type=jnp.float32)
    o_ref[...] = jnp.maximum(acc + b_ref[...], 0.0)


def _grouped_weights(W):
    CG = C // GROUPS
    onehot = (jnp.arange(OUT)[:, None] // (OUT // GROUPS)
              == jnp.arange(GROUPS)[None, :]).astype(W.dtype)
    Wx = (W[:, 0::2][:, None, :] * onehot[:, :, None]).reshape(OUT, C)
    Wm = (W[:, 1::2][:, None, :] * onehot[:, :, None]).reshape(OUT, C)
    del CG
    return Wx, Wm


def kernel(x, edge_index, W, b):
    xT = x.reshape(C, N)
    ep = jnp.transpose(
        edge_index.reshape(2, NBLK, NB, K), (1, 0, 3, 2))

    xb = xT.astype(jnp.bfloat16)
    xpack = lax.bitcast_convert_type(
        jnp.stack([xb[0::2], xb[1::2]], axis=-1), jnp.int32)

    m = _build_sc_aggregate()(xpack, ep)

    Wx, Wm = _grouped_weights(W)
    out = pl.pallas_call(
        _conv_body,
        out_shape=jax.ShapeDtypeStruct((1, OUT, N), jnp.float32),
        out_specs=pl.BlockSpec((None, OUT, N), lambda: (0, 0, 0)),
    )(xT, m, Wx, Wm, b.reshape(OUT, 1))
    return out.reshape(1, OUT, N, 1)


# speedup vs baseline: 1.7088x; 1.7088x over previous
import functools

import jax
import jax.numpy as jnp
from jax import lax
from jax.experimental import pallas as pl
from jax.experimental.pallas import tpu as pltpu
from jax.experimental.pallas import tpu_sc as plsc

C = 128
N = 10000
K = 32
OUT = 128
GROUPS = 4

NC = 2
NS = 16
NW = NC * NS
CPT = C // NW
WPT = CPT // 2
NB = 400
NBLK = N // NB
GPB = NB // 16


@functools.cache
def _build_sc_aggregate():
    mesh = plsc.VectorSubcoreMesh(
        core_axis_name="c", subcore_axis_name="s",
        num_cores=NC, num_subcores=NS)
    return pl.kernel(
        _sc_aggregate_body,
        out_type=jax.ShapeDtypeStruct((C, N), jnp.float32),
        mesh=mesh,
        compiler_params=pltpu.CompilerParams(
            use_tc_tiling_on_sc=False, needs_layout_passes=False),
        scratch_types=[
            pltpu.VMEM((WPT, N), jnp.int32),
            pltpu.VMEM((CPT, N), jnp.float32),
            pltpu.VMEM((2, K, NB), jnp.int32),
            pltpu.VMEM((2, K, NB), jnp.int32),
            pltpu.SemaphoreType.DMA,
            pltpu.SemaphoreType.DMA,
        ],
    )


def _sc_aggregate_body(x_hbm, ep_hbm, m_hbm, x_v, m_v, e_v0, e_v1, sem0, sem1):
    wid = lax.axis_index("s") * NC + lax.axis_index("c")
    c0 = wid * CPT
    pltpu.async_copy(ep_hbm.at[0], e_v0, sem0)
    pltpu.sync_copy(x_hbm.at[pl.ds(wid * WPT, WPT), :], x_v)

    wsplat = [jnp.full((16,), w, jnp.int32) for w in range(WPT)]

    def gather_bf16(widx, nidx):
        word = plsc.load_gather(x_v, [wsplat[widx], nidx])
        return plsc.bitcast(word, jnp.bfloat16)

    def compute_block(blk, e_v):
        @plsc.parallel_loop(0, GPB, unroll=4)
        def g_body(g):
            base = blk * NB + g * 16
            lbase = g * 16

            def edge_vectors(k):
                return (e_v[0, k, pl.ds(lbase, 16)],
                        e_v[1, k, pl.ds(lbase, 16)])

            vj0, vi0 = edge_vectors(0)
            accs = []
            for w in range(WPT):
                accs.append(gather_bf16(w, vj0) - gather_bf16(w, vi0))
            for k in range(1, K):
                vj, vi = edge_vectors(k)
                for w in range(WPT):
                    accs[w] = jnp.maximum(
                        accs[w], gather_bf16(w, vj) - gather_bf16(w, vi))
            for w in range(WPT):
                a_even, a_odd = plsc.unpack(
                    accs[w], format=plsc.PackFormat.INTERLEAVED)
                m_v[2 * w, pl.ds(base, 16)] = a_even
                m_v[2 * w + 1, pl.ds(base, 16)] = a_odd

    def blk_body(blk, _):
        nxt = blk + 1

        @pl.when(blk % 2 == 0)
        def _even():
            @pl.when(nxt < NBLK)
            def _():
                pltpu.async_copy(ep_hbm.at[nxt], e_v1, sem1)
            pltpu.make_async_copy(ep_hbm.at[blk], e_v0, sem0).wait()
            compute_block(blk, e_v0)

        @pl.when(blk % 2 == 1)
        def _odd():
            @pl.when(nxt < NBLK)
            def _():
                pltpu.async_copy(ep_hbm.at[nxt], e_v0, sem0)
            pltpu.make_async_copy(ep_hbm.at[blk], e_v1, sem1).wait()
            compute_block(blk, e_v1)

        return 0

    lax.fori_loop(0, NBLK, blk_body, 0)
    pltpu.sync_copy(m_v, m_hbm.at[pl.ds(c0, CPT), :])


def _conv_body(x_ref, m_ref, wx_ref, wm_ref, b_ref, o_ref):
    acc = jnp.dot(wx_ref[...], x_ref[...], preferred_element_type=jnp.float32)
    acc = acc + jnp.dot(wm_ref[...], m_ref[...], preferred_element_type=jnp.float32)
    o_ref[...] = jnp.maximum(acc + b_ref[...], 0.0)


def _grouped_weights(W):
    CG = C // GROUPS
    onehot = (jnp.arange(OUT)[:, None] // (OUT // GROUPS)
              == jnp.arange(GROUPS)[None, :]).astype(W.dtype)
    Wx = (W[:, 0::2][:, None, :] * onehot[:, :, None]).reshape(OUT, C)
    Wm = (W[:, 1::2][:, None, :] * onehot[:, :, None]).reshape(OUT, C)
    del CG
    return Wx, Wm


def kernel(x, edge_index, W, b):
    xT = x.reshape(C, N)
    ep = jnp.transpose(
        edge_index.reshape(2, NBLK, NB, K), (1, 0, 3, 2))

    xb = xT.astype(jnp.bfloat16)
    xpack = lax.bitcast_convert_type(
        jnp.stack([xb[0::2], xb[1::2]], axis=-1), jnp.int32)

    m = _build_sc_aggregate()(xpack, ep)

    Wx, Wm = _grouped_weights(W)
    out = pl.pallas_call(
        _conv_body,
        out_shape=jax.ShapeDtypeStruct((1, OUT, N), jnp.float32),
        out_specs=pl.BlockSpec((None, OUT, N), lambda: (0, 0, 0)),
    )(xT, m, Wx, Wm, b.reshape(OUT, 1))
    return out.reshape(1, OUT, N, 1)


# speedup vs baseline: 1.7706x; 1.0362x over previous
import functools

import jax
import jax.numpy as jnp
from jax import lax
from jax.experimental import pallas as pl
from jax.experimental.pallas import tpu as pltpu
from jax.experimental.pallas import tpu_sc as plsc

C = 128
N = 10000
K = 32
OUT = 128
GROUPS = 4

NC = 2
NS = 16
NW = NC * NS
CPT = C // NW
WPT = CPT // 2
NB = 400
NBLK = N // NB
GPB = NB // 16


@functools.cache
def _build_sc_aggregate():
    mesh = plsc.VectorSubcoreMesh(
        core_axis_name="c", subcore_axis_name="s",
        num_cores=NC, num_subcores=NS)
    return pl.kernel(
        _sc_aggregate_body,
        out_type=jax.ShapeDtypeStruct((C, N), jnp.float32),
        mesh=mesh,
        compiler_params=pltpu.CompilerParams(
            use_tc_tiling_on_sc=False, needs_layout_passes=False),
        scratch_types=[
            pltpu.VMEM((WPT, N), jnp.int32),
            pltpu.VMEM((CPT, N), jnp.float32),
            pltpu.VMEM((2, K, NB), jnp.int32),
            pltpu.VMEM((2, K, NB), jnp.int32),
            pltpu.SemaphoreType.DMA,
            pltpu.SemaphoreType.DMA,
        ],
    )


def _sc_aggregate_body(x_hbm, ep_hbm, m_hbm, x_v, m_v, e_v0, e_v1, sem0, sem1):
    wid = lax.axis_index("s") * NC + lax.axis_index("c")
    c0 = wid * CPT
    pltpu.async_copy(ep_hbm.at[:, :, pl.ds(0, NB)], e_v0, sem0)
    pltpu.sync_copy(x_hbm.at[pl.ds(wid * WPT, WPT), :], x_v)

    wsplat = [jnp.full((16,), w, jnp.int32) for w in range(WPT)]

    def gather_bf16(widx, nidx):
        word = plsc.load_gather(x_v, [wsplat[widx], nidx])
        return plsc.bitcast(word, jnp.bfloat16)

    def compute_block(blk, e_v):
        @plsc.parallel_loop(0, GPB, unroll=4)
        def g_body(g):
            base = blk * NB + g * 16
            lbase = g * 16

            def edge_vectors(k):
                return (e_v[0, k, pl.ds(lbase, 16)],
                        e_v[1, k, pl.ds(lbase, 16)])

            vj0, vi0 = edge_vectors(0)
            accs = []
            for w in range(WPT):
                accs.append(gather_bf16(w, vj0) - gather_bf16(w, vi0))
            for k in range(1, K):
                vj, vi = edge_vectors(k)
                for w in range(WPT):
                    accs[w] = jnp.maximum(
                        accs[w], gather_bf16(w, vj) - gather_bf16(w, vi))
            for w in range(WPT):
                a_even, a_odd = plsc.unpack(
                    accs[w], format=plsc.PackFormat.INTERLEAVED)
                m_v[2 * w, pl.ds(base, 16)] = a_even
                m_v[2 * w + 1, pl.ds(base, 16)] = a_odd

    def blk_body(blk, _):
        nxt = blk + 1

        @pl.when(blk % 2 == 0)
        def _even():
            @pl.when(nxt < NBLK)
            def _():
                pltpu.async_copy(
                    ep_hbm.at[:, :, pl.ds(nxt * NB, NB)], e_v1, sem1)
            pltpu.make_async_copy(
                ep_hbm.at[:, :, pl.ds(blk * NB, NB)], e_v0, sem0).wait()
            compute_block(blk, e_v0)

        @pl.when(blk % 2 == 1)
        def _odd():
            @pl.when(nxt < NBLK)
            def _():
                pltpu.async_copy(
                    ep_hbm.at[:, :, pl.ds(nxt * NB, NB)], e_v0, sem0)
            pltpu.make_async_copy(
                ep_hbm.at[:, :, pl.ds(blk * NB, NB)], e_v1, sem1).wait()
            compute_block(blk, e_v1)

        return 0

    lax.fori_loop(0, NBLK, blk_body, 0)
    pltpu.sync_copy(m_v, m_hbm.at[pl.ds(c0, CPT), :])


def _conv_body(x_ref, m_ref, wx_ref, wm_ref, b_ref, o_ref):
    acc = jnp.dot(wx_ref[...], x_ref[...], preferred_element_type=jnp.float32)
    acc = acc + jnp.dot(wm_ref[...], m_ref[...], preferred_element_type=jnp.float32)
    o_ref[...] = jnp.maximum(acc + b_ref[...], 0.0)


def _grouped_weights(W):
    CG = C // GROUPS
    onehot = (jnp.arange(OUT)[:, None] // (OUT // GROUPS)
              == jnp.arange(GROUPS)[None, :]).astype(W.dtype)
    Wx = (W[:, 0::2][:, None, :] * onehot[:, :, None]).reshape(OUT, C)
    Wm = (W[:, 1::2][:, None, :] * onehot[:, :, None]).reshape(OUT, C)
    del CG
    return Wx, Wm


def kernel(x, edge_index, W, b):
    xT = x.reshape(C, N)
    ep = jnp.transpose(edge_index.reshape(2, N, K), (0, 2, 1))

    xb = xT.astype(jnp.bfloat16)
    xpack = lax.bitcast_convert_type(
        jnp.stack([xb[0::2], xb[1::2]], axis=-1), jnp.int32)

    m = _build_sc_aggregate()(xpack, ep)

    Wx, Wm = _grouped_weights(W)
    out = pl.pallas_call(
        _conv_body,
        out_shape=jax.ShapeDtypeStruct((1, OUT, N), jnp.float32),
        out_specs=pl.BlockSpec((None, OUT, N), lambda: (0, 0, 0)),
    )(xT, m, Wx, Wm, b.reshape(OUT, 1))
    return out.reshape(1, OUT, N, 1)
